# Initial kernel scaffold; baseline (speedup 1.0000x reference)
import jax
import jax.numpy as jnp
from jax.experimental import pallas as pl


def kernel(nodes_real, to_neighs, features):
    raise NotImplementedError("write your pallas kernel here")


# speedup vs baseline: 1.6176x; 1.6176x over previous
import functools

import jax
import jax.numpy as jnp
from jax import lax
from jax.experimental import pallas as pl
from jax.experimental.pallas import tpu as pltpu
from jax.experimental.pallas import tpu_sc as plsc

_B = 10000
_DEG = 32
_D = 128
_NC = 2
_NS = 16
_NW = _NC * _NS
_RPW = 320
_B_PAD = _NW * _RPW
_NBUF = 4
_NG = _RPW // _NBUF


def _weights_body(xt_ref, wt_ref):
    x = xt_ref[...]
    rows = lax.broadcasted_iota(jnp.int32, x.shape, 0)
    dup = jnp.zeros(x.shape, jnp.bool_)
    for k in range(_DEG - 1):
        dup = jnp.logical_or(
            dup, jnp.logical_and(x == x[k : k + 1, :], rows > k)
        )
    first = jnp.logical_not(dup).astype(jnp.float32)
    u = jnp.sum(first, axis=0, keepdims=True)
    wt_ref[...] = first / u


def _weights_tc(xt):
    return pl.pallas_call(
        _weights_body,
        out_shape=jax.ShapeDtypeStruct((_DEG, _B_PAD), jnp.float32),
    )(xt)


@functools.partial(
    pl.kernel,
    out_type=jax.ShapeDtypeStruct((_B_PAD, _D), jnp.float32),
    mesh=plsc.VectorSubcoreMesh(core_axis_name="c", subcore_axis_name="s"),
    scratch_types=[
        pltpu.VMEM((_RPW, _DEG), jnp.int32),
        pltpu.VMEM((_RPW, _DEG), jnp.float32),
        pltpu.VMEM((_NBUF, _D), jnp.float32),
        pltpu.VMEM((_NBUF, _DEG, _D), jnp.float32),
        pltpu.SemaphoreType.DMA,
        pltpu.SemaphoreType.DMA,
        pltpu.SemaphoreType.DMA,
        pltpu.SemaphoreType.DMA,
        pltpu.SemaphoreType.DMA,
        pltpu.SemaphoreType.DMA,
        pltpu.SemaphoreType.DMA,
        pltpu.SemaphoreType.DMA,
    ],
)
def _sc_aggregate(idx_hbm, w_hbm, feat_hbm, out_hbm,
                  idx_v, w_v, obuf, gbuf, *sems):
    gsems, osems = sems[:_NBUF], sems[_NBUF:]
    wid = lax.axis_index("s") * _NC + lax.axis_index("c")
    base = wid * _RPW
    pltpu.sync_copy(idx_hbm.at[pl.ds(base, _RPW)], idx_v)
    pltpu.sync_copy(w_hbm.at[pl.ds(base, _RPW)], w_v)

    def _gather(row, b):
        return pltpu.make_async_copy(
            feat_hbm.at[idx_v.at[row]], gbuf.at[b], gsems[b]
        )

    def _put(row, b):
        return pltpu.make_async_copy(
            obuf.at[b], out_hbm.at[base + row], osems[b]
        )

    for b in range(_NBUF):
        _gather(b, b).start()

    def body(g, carry):
        for b in range(_NBUF):
            row = g * _NBUF + b
            _gather(row, b).wait()
            wa = w_v[row, pl.ds(0, 16)]
            wb = w_v[row, pl.ds(16, 16)]
            ws = [wa[j] for j in range(16)] + [wb[j] for j in range(16)]

            @pl.when(g > 0)
            def _():
                _put(row - _NBUF, b).wait()

            for d in range(_D // 16):
                acc = gbuf[b, 0, pl.ds(d * 16, 16)] * ws[0]
                for j in range(1, _DEG):
                    acc = acc + gbuf[b, j, pl.ds(d * 16, 16)] * ws[j]
                obuf[b, pl.ds(d * 16, 16)] = acc
            _put(row, b).start()
            nxt = row + _NBUF

            @pl.when(nxt < _RPW)
            def _():
                _gather(nxt, b).start()

        return carry

    lax.fori_loop(0, _NG, body, 0)
    for b in range(_NBUF):
        _put(_RPW - _NBUF + b, b).wait()


def kernel(nodes_real, to_neighs, features):
    del nodes_real
    idx_pad = jnp.pad(to_neighs, ((0, _B_PAD - _B), (0, 0)))
    w = _weights_tc(idx_pad.T).T
    out = _sc_aggregate(idx_pad, w, features)
    return out[:_B]
